# baseline (speedup 1.0000x reference)
import functools

import jax
import jax.numpy as jnp
from jax import lax
from jax.experimental import pallas as pl
from jax.experimental.pallas import tpu as pltpu
from jax.experimental.pallas import tpu_sc as plsc

N = 10000
E = 320000
D = 128
EPS = 1e-5
NP = N + 8

NC = 2
NS = 16
NW = NC * NS
EPW = E // NW
CH = 112
NCH = 90
EPWP = NCH * CH
NBUF = 3
ACCN = 10240
RPW = ACCN // NS

_mesh = plsc.VectorSubcoreMesh(
    core_axis_name="c", subcore_axis_name="s", num_cores=NC, num_subcores=NS
)

_sc_params = pltpu.CompilerParams(needs_layout_passes=False)


@functools.partial(
    pl.kernel,
    mesh=_mesh,
    out_type=jax.ShapeDtypeStruct((NW, N), jnp.float32),
    scratch_types=[
        pltpu.VMEM((EPW,), jnp.int32),
        pltpu.VMEM((EPW,), jnp.int32),
        pltpu.VMEM((N,), jnp.float32),
    ],
    compiler_params=_sc_params,
)
def _deg_kernel(src_hbm, dst_hbm, out_hbm, src_v, dst_v, acc_v):
    cid = lax.axis_index("c")
    sid = lax.axis_index("s")
    wid = sid * NC + cid
    base = wid * EPW
    pltpu.sync_copy(src_hbm.at[pl.ds(base, EPW)], src_v)
    pltpu.sync_copy(dst_hbm.at[pl.ds(base, EPW)], dst_v)

    zeros16 = jnp.zeros((16,), jnp.float32)

    def zero_body(i, carry):
        acc_v[pl.ds(i * 16, 16)] = zeros16
        return carry

    lax.fori_loop(0, N // 16, zero_body, 0)

    def edge_body(i, carry):
        s = src_v[pl.ds(i * 16, 16)]
        d = dst_v[pl.ds(i * 16, 16)]
        w = jnp.where(s != d, 1.0, 0.0).astype(jnp.float32)
        plsc.addupdate_scatter(acc_v, [s], w)
        return carry

    lax.fori_loop(0, EPW // 16, edge_body, 0)
    pltpu.sync_copy(acc_v, out_hbm.at[wid])


@functools.partial(
    pl.kernel,
    mesh=_mesh,
    out_type=jax.ShapeDtypeStruct((NC, ACCN, D), jnp.float32),
    scratch_types=[
        [pltpu.VMEM((2, CH), jnp.int32)] * NBUF,
        [pltpu.VMEM((2, CH), jnp.int32)] * NBUF,
        [pltpu.VMEM((CH, D), jnp.float32)] * NBUF,
        pltpu.VMEM_SHARED((ACCN, D), jnp.float32),
        [pltpu.SemaphoreType.DMA] * NBUF,
        [pltpu.SemaphoreType.DMA] * NBUF,
        [pltpu.SemaphoreType.DMA] * NBUF,
    ],
    compiler_params=_sc_params,
)
def _prop_kernel(u_hbm, idx_hbm, zrows_hbm, out_hbm,
                 idxa_v, idxb_v, rows_v, acc_sh, isema, isemb, gsem):
    cid = lax.axis_index("c")
    sid = lax.axis_index("s")
    wid = sid * NC + cid

    pltpu.sync_copy(zrows_hbm, rows_v[0])
    for j in range(RPW // CH):
        pltpu.sync_copy(rows_v[0], acc_sh.at[pl.ds(sid * RPW + j * CH, CH)])
    rem = RPW - (RPW // CH) * CH
    if rem:
        pltpu.sync_copy(
            rows_v[0].at[pl.ds(0, rem)],
            acc_sh.at[pl.ds(sid * RPW + (RPW // CH) * CH, rem)])
    plsc.subcore_barrier()

    NR = NCH // NBUF

    def run_round(t, cur, csem, nxt, nsem, prefetch):
        gdesc = []
        for b in range(NBUF):
            pltpu.make_async_copy(
                idx_hbm.at[wid, t * NBUF + b], cur[b], csem[b]).wait()
            gdesc.append(
                pltpu.async_copy(u_hbm.at[cur[b].at[0]], rows_v[b], gsem[b]))
        if prefetch is not None:
            for b in range(NBUF):
                pltpu.async_copy(
                    idx_hbm.at[wid, prefetch * NBUF + b], nxt[b], nsem[b])
        for b in range(NBUF):
            gdesc[b].wait()
            pltpu.sync_copy(rows_v[b], acc_sh.at[cur[b].at[1]], add=True)

    for b in range(NBUF):
        pltpu.async_copy(idx_hbm.at[wid, b], idxa_v[b], isema[b])

    def pair_body(p, carry):
        t0 = 2 * p
        run_round(t0, idxa_v, isema, idxb_v, isemb, t0 + 1)

        @pl.when(t0 + 2 < NR)
        def _():
            for b in range(NBUF):
                pltpu.async_copy(
                    idx_hbm.at[wid, (t0 + 2) * NBUF + b], idxa_v[b], isema[b])
        run_round(t0 + 1, idxb_v, isemb, idxa_v, isema, None)
        return carry

    lax.fori_loop(0, NR // 2, pair_body, 0)
    plsc.subcore_barrier()

    off = sid * RPW
    pltpu.sync_copy(acc_sh.at[pl.ds(off, RPW)],
                    out_hbm.at[cid, pl.ds(off, RPW)])


def _tcA_body(degp_ref, x_ref, u0_ref, dinv_ref):
    deg = jnp.sum(degp_ref[...], axis=0)
    dinv = jnp.where(deg > 0.0, lax.rsqrt(jnp.maximum(deg, 1.0)), 0.0)
    dv = dinv[:, None]
    dinv_ref[...] = dv
    u0_ref[pl.ds(0, N), :] = x_ref[...] * dv
    u0_ref[pl.ds(N, NP - N), :] = jnp.zeros((NP - N, D), jnp.float32)


def _tcB_body(s1_ref, dinv_ref, tx1_ref, u1_ref):
    s = s1_ref[0, pl.ds(0, N), :] + s1_ref[1, pl.ds(0, N), :]
    dv = dinv_ref[...]
    tx1 = -(dv * s)
    tx1_ref[...] = tx1
    u1_ref[pl.ds(0, N), :] = dv * tx1
    u1_ref[pl.ds(N, NP - N), :] = jnp.zeros((NP - N, D), jnp.float32)


def _tcC1_body(x_ref, tx1_ref, w_ref, b_ref, part_ref):
    out = jnp.dot(x_ref[...], w_ref[0], preferred_element_type=jnp.float32)
    out += jnp.dot(tx1_ref[...], w_ref[1], preferred_element_type=jnp.float32)
    part_ref[...] = out + b_ref[...]


def _tcC2_body(x_ref, part_ref, s2_ref, dinv_ref, w_ref, g_ref, be_ref,
               y_ref):
    x = x_ref[...]
    dv = dinv_ref[...]
    s2 = s2_ref[0, pl.ds(0, N), :] + s2_ref[1, pl.ds(0, N), :]
    tx2 = -2.0 * (dv * s2) - x
    out = part_ref[...] + jnp.dot(tx2, w_ref[2],
                                  preferred_element_type=jnp.float32)
    mean = jnp.mean(out, axis=0, keepdims=True)
    var = jnp.mean((out - mean) ** 2, axis=0, keepdims=True)
    y_ref[...] = (out - mean) * lax.rsqrt(var + EPS) * g_ref[...] + be_ref[...]


_tcA = pl.pallas_call(
    _tcA_body,
    out_shape=(
        jax.ShapeDtypeStruct((NP, D), jnp.float32),
        jax.ShapeDtypeStruct((N, 1), jnp.float32),
    ),
)

_tcB = pl.pallas_call(
    _tcB_body,
    out_shape=(
        jax.ShapeDtypeStruct((N, D), jnp.float32),
        jax.ShapeDtypeStruct((NP, D), jnp.float32),
    ),
)

_tcC1 = pl.pallas_call(
    _tcC1_body,
    out_shape=jax.ShapeDtypeStruct((N, D), jnp.float32),
)

_tcC2 = pl.pallas_call(
    _tcC2_body,
    out_shape=jax.ShapeDtypeStruct((N, D), jnp.float32),
)


def kernel(x, edge_index, W, b, gamma, beta):
    src = edge_index[0]
    dst = edge_index[1]
    srcp = jnp.where(src == dst, N, src)

    npad = EPWP - EPW
    srcp_p = jnp.concatenate(
        [srcp.reshape(NW, EPW), jnp.full((NW, npad), N, jnp.int32)], axis=1
    ).reshape(NW, NCH, 1, CH)
    dst_p = jnp.concatenate(
        [dst.reshape(NW, EPW), jnp.zeros((NW, npad), jnp.int32)], axis=1
    ).reshape(NW, NCH, 1, CH)
    idx4 = jnp.concatenate([srcp_p, dst_p], axis=2)
    zrows = jnp.zeros((CH, D), jnp.float32)

    degp = _deg_kernel(src, dst)
    u0, dinv = _tcA(degp, x)
    s1 = _prop_kernel(u0, idx4, zrows)
    tx1, u1 = _tcB(s1, dinv)
    s2 = _prop_kernel(u1, idx4, zrows)
    part = _tcC1(x, tx1, W, b.reshape(1, D))
    return _tcC2(x, part, s2, dinv,
                 W, gamma.reshape(1, D), beta.reshape(1, D))


# speedup vs baseline: 1.1174x; 1.1174x over previous
import functools

import jax
import jax.numpy as jnp
from jax import lax
from jax.experimental import pallas as pl
from jax.experimental.pallas import tpu as pltpu
from jax.experimental.pallas import tpu_sc as plsc

N = 10000
E = 320000
D = 128
EPS = 1e-5
NP = N + 8

NC = 2
NS = 16
NW = NC * NS
EPW = E // NW
CH = 112
NCH = 90
EPWP = NCH * CH
NBUF = 3
ACCN = 10240
RPW = ACCN // NS

_mesh = plsc.VectorSubcoreMesh(
    core_axis_name="c", subcore_axis_name="s", num_cores=NC, num_subcores=NS
)

_sc_params = pltpu.CompilerParams(needs_layout_passes=False)


@functools.partial(
    pl.kernel,
    mesh=_mesh,
    out_type=jax.ShapeDtypeStruct((NW, N), jnp.float32),
    scratch_types=[
        pltpu.VMEM((EPW,), jnp.int32),
        pltpu.VMEM((EPW,), jnp.int32),
        pltpu.VMEM((N,), jnp.float32),
    ],
    compiler_params=_sc_params,
)
def _deg_kernel(src_hbm, dst_hbm, out_hbm, src_v, dst_v, acc_v):
    cid = lax.axis_index("c")
    sid = lax.axis_index("s")
    wid = sid * NC + cid
    base = wid * EPW
    pltpu.sync_copy(src_hbm.at[pl.ds(base, EPW)], src_v)
    pltpu.sync_copy(dst_hbm.at[pl.ds(base, EPW)], dst_v)

    zeros16 = jnp.zeros((16,), jnp.float32)

    def zero_body(i, carry):
        acc_v[pl.ds(i * 16, 16)] = zeros16
        return carry

    lax.fori_loop(0, N // 16, zero_body, 0)

    def edge_body(i, carry):
        s = src_v[pl.ds(i * 16, 16)]
        d = dst_v[pl.ds(i * 16, 16)]
        w = jnp.where(s != d, 1.0, 0.0).astype(jnp.float32)
        plsc.addupdate_scatter(acc_v, [s], w)
        return carry

    lax.fori_loop(0, EPW // 16, edge_body, 0)
    pltpu.sync_copy(acc_v, out_hbm.at[wid])


@functools.partial(
    pl.kernel,
    mesh=_mesh,
    out_type=jax.ShapeDtypeStruct((NC, ACCN, D), jnp.float32),
    scratch_types=[
        [pltpu.VMEM((2, CH), jnp.int32)] * NBUF,
        [pltpu.VMEM((2, CH), jnp.int32)] * NBUF,
        [pltpu.VMEM((CH, D), jnp.float32)] * NBUF,
        pltpu.VMEM_SHARED((ACCN, D), jnp.float32),
        [pltpu.SemaphoreType.DMA] * NBUF,
        [pltpu.SemaphoreType.DMA] * NBUF,
        [pltpu.SemaphoreType.DMA] * NBUF,
        [pltpu.SemaphoreType.DMA] * NBUF,
    ],
    compiler_params=_sc_params,
)
def _prop_kernel(u_hbm, idx_hbm, zrows_hbm, out_hbm,
                 idxa_v, idxb_v, rows_v, acc_sh, isema, isemb, gsem, ssem):
    cid = lax.axis_index("c")
    sid = lax.axis_index("s")
    wid = sid * NC + cid
    NR = NCH // NBUF

    for b in range(NBUF):
        pltpu.async_copy(idx_hbm.at[wid, b], idxa_v[b], isema[b])
        pltpu.async_copy(idx_hbm.at[wid, NBUF + b], idxb_v[b], isemb[b])

    pltpu.sync_copy(zrows_hbm, rows_v[0])
    for j in range(RPW // CH):
        pltpu.sync_copy(rows_v[0], acc_sh.at[pl.ds(sid * RPW + j * CH, CH)])
    rem = RPW - (RPW // CH) * CH
    if rem:
        pltpu.sync_copy(
            rows_v[0].at[pl.ds(0, rem)],
            acc_sh.at[pl.ds(sid * RPW + (RPW // CH) * CH, rem)])

    for b in range(NBUF):
        pltpu.make_async_copy(idx_hbm.at[wid, b], idxa_v[b], isema[b]).wait()
        pltpu.async_copy(u_hbm.at[idxa_v[b].at[0]], rows_v[b], gsem[b])
    plsc.subcore_barrier()

    def finish_launch(t, cur, csem, nxt, nsem):
        for b in range(NBUF):
            pltpu.make_async_copy(
                u_hbm.at[cur[b].at[0]], rows_v[b], gsem[b]).wait()
            pltpu.async_copy(rows_v[b], acc_sh.at[cur[b].at[1]], ssem[b],
                             add=True)
        for b in range(NBUF):
            pltpu.make_async_copy(
                rows_v[b], acc_sh.at[cur[b].at[1]], ssem[b]).wait()

            @pl.when(t + 2 < NR)
            def _():
                pltpu.async_copy(
                    idx_hbm.at[wid, (t + 2) * NBUF + b], cur[b], csem[b])

            @pl.when(t + 1 < NR)
            def _():
                pltpu.make_async_copy(
                    idx_hbm.at[wid, (t + 1) * NBUF + b], nxt[b],
                    nsem[b]).wait()
                pltpu.async_copy(u_hbm.at[nxt[b].at[0]], rows_v[b], gsem[b])

    def pair_body(p, carry):
        finish_launch(2 * p, idxa_v, isema, idxb_v, isemb)
        finish_launch(2 * p + 1, idxb_v, isemb, idxa_v, isema)
        return carry

    lax.fori_loop(0, NR // 2, pair_body, 0)
    plsc.subcore_barrier()

    off = sid * RPW
    pltpu.sync_copy(acc_sh.at[pl.ds(off, RPW)],
                    out_hbm.at[cid, pl.ds(off, RPW)])


def _tcA_body(degp_ref, x_ref, u0_ref, dinv_ref):
    deg = jnp.sum(degp_ref[...], axis=0)
    dinv = jnp.where(deg > 0.0, lax.rsqrt(jnp.maximum(deg, 1.0)), 0.0)
    dv = dinv[:, None]
    dinv_ref[...] = dv
    u0_ref[pl.ds(0, N), :] = x_ref[...] * dv
    u0_ref[pl.ds(N, NP - N), :] = jnp.zeros((NP - N, D), jnp.float32)


def _tcB_body(s1_ref, dinv_ref, tx1_ref, u1_ref):
    s = s1_ref[0, pl.ds(0, N), :] + s1_ref[1, pl.ds(0, N), :]
    dv = dinv_ref[...]
    tx1 = -(dv * s)
    tx1_ref[...] = tx1
    u1_ref[pl.ds(0, N), :] = dv * tx1
    u1_ref[pl.ds(N, NP - N), :] = jnp.zeros((NP - N, D), jnp.float32)


def _tcC1_body(x_ref, tx1_ref, w_ref, b_ref, part_ref):
    out = jnp.dot(x_ref[...], w_ref[0], preferred_element_type=jnp.float32)
    out += jnp.dot(tx1_ref[...], w_ref[1], preferred_element_type=jnp.float32)
    part_ref[...] = out + b_ref[...]


def _tcC2_body(x_ref, part_ref, s2_ref, dinv_ref, w_ref, g_ref, be_ref,
               y_ref):
    x = x_ref[...]
    dv = dinv_ref[...]
    s2 = s2_ref[0, pl.ds(0, N), :] + s2_ref[1, pl.ds(0, N), :]
    tx2 = -2.0 * (dv * s2) - x
    out = part_ref[...] + jnp.dot(tx2, w_ref[2],
                                  preferred_element_type=jnp.float32)
    mean = jnp.mean(out, axis=0, keepdims=True)
    var = jnp.mean((out - mean) ** 2, axis=0, keepdims=True)
    y_ref[...] = (out - mean) * lax.rsqrt(var + EPS) * g_ref[...] + be_ref[...]


_tcA = pl.pallas_call(
    _tcA_body,
    out_shape=(
        jax.ShapeDtypeStruct((NP, D), jnp.float32),
        jax.ShapeDtypeStruct((N, 1), jnp.float32),
    ),
)

_tcB = pl.pallas_call(
    _tcB_body,
    out_shape=(
        jax.ShapeDtypeStruct((N, D), jnp.float32),
        jax.ShapeDtypeStruct((NP, D), jnp.float32),
    ),
)

_tcC1 = pl.pallas_call(
    _tcC1_body,
    out_shape=jax.ShapeDtypeStruct((N, D), jnp.float32),
)

_tcC2 = pl.pallas_call(
    _tcC2_body,
    out_shape=jax.ShapeDtypeStruct((N, D), jnp.float32),
)


def kernel(x, edge_index, W, b, gamma, beta):
    src = edge_index[0]
    dst = edge_index[1]
    srcp = jnp.where(src == dst, N, src)

    npad = EPWP - EPW
    srcp_p = jnp.concatenate(
        [srcp.reshape(NW, EPW), jnp.full((NW, npad), N, jnp.int32)], axis=1
    ).reshape(NW, NCH, 1, CH)
    dst_p = jnp.concatenate(
        [dst.reshape(NW, EPW), jnp.zeros((NW, npad), jnp.int32)], axis=1
    ).reshape(NW, NCH, 1, CH)
    idx4 = jnp.concatenate([srcp_p, dst_p], axis=2)
    zrows = jnp.zeros((CH, D), jnp.float32)

    degp = _deg_kernel(src, dst)
    u0, dinv = _tcA(degp, x)
    s1 = _prop_kernel(u0, idx4, zrows)
    tx1, u1 = _tcB(s1, dinv)
    s2 = _prop_kernel(u1, idx4, zrows)
    part = _tcC1(x, tx1, W, b.reshape(1, D))
    return _tcC2(x, part, s2, dinv,
                 W, gamma.reshape(1, D), beta.reshape(1, D))


# speedup vs baseline: 1.2022x; 1.0759x over previous
import functools

import jax
import jax.numpy as jnp
from jax import lax
from jax.experimental import pallas as pl
from jax.experimental.pallas import tpu as pltpu
from jax.experimental.pallas import tpu_sc as plsc

N = 10000
E = 320000
D = 128
EPS = 1e-5
NP = N + 8

NC = 2
NS = 16
NW = NC * NS
EPW = E // NW
CH = 112
NCH = 90
EPWP = NCH * CH
NBUF = 3
ACCN = 10240
RPW = ACCN // NS

_mesh = plsc.VectorSubcoreMesh(
    core_axis_name="c", subcore_axis_name="s", num_cores=NC, num_subcores=NS
)

_sc_params = pltpu.CompilerParams(needs_layout_passes=False)


@functools.partial(
    pl.kernel,
    mesh=_mesh,
    out_type=jax.ShapeDtypeStruct((NW, N), jnp.float32),
    scratch_types=[
        pltpu.VMEM((EPW,), jnp.int32),
        pltpu.VMEM((EPW,), jnp.int32),
        pltpu.VMEM((N,), jnp.float32),
    ],
    compiler_params=_sc_params,
)
def _deg_kernel(src_hbm, dst_hbm, out_hbm, src_v, dst_v, acc_v):
    cid = lax.axis_index("c")
    sid = lax.axis_index("s")
    wid = sid * NC + cid
    base = wid * EPW
    pltpu.sync_copy(src_hbm.at[pl.ds(base, EPW)], src_v)
    pltpu.sync_copy(dst_hbm.at[pl.ds(base, EPW)], dst_v)

    zeros16 = jnp.zeros((16,), jnp.float32)

    def zero_body(i, carry):
        acc_v[pl.ds(i * 16, 16)] = zeros16
        return carry

    lax.fori_loop(0, N // 16, zero_body, 0)

    def edge_body(i, carry):
        s = src_v[pl.ds(i * 16, 16)]
        d = dst_v[pl.ds(i * 16, 16)]
        w = jnp.where(s != d, 1.0, 0.0).astype(jnp.float32)
        plsc.addupdate_scatter(acc_v, [s], w)
        return carry

    lax.fori_loop(0, EPW // 16, edge_body, 0)
    pltpu.sync_copy(acc_v, out_hbm.at[wid])


@functools.partial(
    pl.kernel,
    mesh=_mesh,
    out_type=jax.ShapeDtypeStruct((NC, ACCN, D), jnp.float32),
    scratch_types=[
        [pltpu.VMEM((2, CH), jnp.int32)] * NBUF,
        [pltpu.VMEM((2, CH), jnp.int32)] * NBUF,
        [pltpu.VMEM((CH, D), jnp.float32)] * NBUF,
        pltpu.VMEM_SHARED((ACCN, D), jnp.float32),
        [pltpu.SemaphoreType.DMA] * NBUF,
        [pltpu.SemaphoreType.DMA] * NBUF,
        [pltpu.SemaphoreType.DMA] * NBUF,
        [pltpu.SemaphoreType.DMA] * NBUF,
    ],
    compiler_params=_sc_params,
)
def _prop_kernel(u_hbm, idx_hbm, zrows_hbm, out_hbm,
                 idxa_v, idxb_v, rows_v, acc_sh, isema, isemb, gsem, ssem):
    cid = lax.axis_index("c")
    sid = lax.axis_index("s")
    wid = sid * NC + cid
    NR = NCH // NBUF

    for b in range(NBUF):
        pltpu.async_copy(idx_hbm.at[wid, b], idxa_v[b], isema[b])
        pltpu.async_copy(idx_hbm.at[wid, NBUF + b], idxb_v[b], isemb[b])

    pltpu.sync_copy(zrows_hbm, rows_v[0])
    for j in range(RPW // CH):
        pltpu.sync_copy(rows_v[0], acc_sh.at[pl.ds(sid * RPW + j * CH, CH)])
    rem = RPW - (RPW // CH) * CH
    if rem:
        pltpu.sync_copy(
            rows_v[0].at[pl.ds(0, rem)],
            acc_sh.at[pl.ds(sid * RPW + (RPW // CH) * CH, rem)])

    for b in range(NBUF):
        pltpu.make_async_copy(idx_hbm.at[wid, b], idxa_v[b], isema[b]).wait()
        pltpu.async_copy(u_hbm.at[idxa_v[b].at[0]], rows_v[b], gsem[b])
    plsc.subcore_barrier()

    def finish_launch(t, cur, csem, nxt, nsem):
        for b in range(NBUF):
            pltpu.make_async_copy(
                u_hbm.at[cur[b].at[0]], rows_v[b], gsem[b]).wait()
            pltpu.sync_copy(rows_v[b], acc_sh.at[cur[b].at[1]], add=True)

            @pl.when(t + 2 < NR)
            def _():
                pltpu.async_copy(
                    idx_hbm.at[wid, (t + 2) * NBUF + b], cur[b], csem[b])

            @pl.when(t + 1 < NR)
            def _():
                pltpu.make_async_copy(
                    idx_hbm.at[wid, (t + 1) * NBUF + b], nxt[b],
                    nsem[b]).wait()
                pltpu.async_copy(u_hbm.at[nxt[b].at[0]], rows_v[b], gsem[b])

    def pair_body(p, carry):
        finish_launch(2 * p, idxa_v, isema, idxb_v, isemb)
        finish_launch(2 * p + 1, idxb_v, isemb, idxa_v, isema)
        return carry

    lax.fori_loop(0, NR // 2, pair_body, 0)
    plsc.subcore_barrier()

    off = sid * RPW
    pltpu.sync_copy(acc_sh.at[pl.ds(off, RPW)],
                    out_hbm.at[cid, pl.ds(off, RPW)])


def _tcA_body(degp_ref, x_ref, u0_ref, dinv_ref):
    deg = jnp.sum(degp_ref[...], axis=0)
    dinv = jnp.where(deg > 0.0, lax.rsqrt(jnp.maximum(deg, 1.0)), 0.0)
    dv = dinv[:, None]
    dinv_ref[...] = dv
    u0_ref[pl.ds(0, N), :] = x_ref[...] * dv
    u0_ref[pl.ds(N, NP - N), :] = jnp.zeros((NP - N, D), jnp.float32)


def _tcB_body(s1_ref, dinv_ref, tx1_ref, u1_ref):
    s = s1_ref[0, pl.ds(0, N), :] + s1_ref[1, pl.ds(0, N), :]
    dv = dinv_ref[...]
    tx1 = -(dv * s)
    tx1_ref[...] = tx1
    u1_ref[pl.ds(0, N), :] = dv * tx1
    u1_ref[pl.ds(N, NP - N), :] = jnp.zeros((NP - N, D), jnp.float32)


def _tcC1_body(x_ref, tx1_ref, w_ref, b_ref, part_ref):
    out = jnp.dot(x_ref[...], w_ref[0], preferred_element_type=jnp.float32)
    out += jnp.dot(tx1_ref[...], w_ref[1], preferred_element_type=jnp.float32)
    part_ref[...] = out + b_ref[...]


def _tcC2_body(x_ref, part_ref, s2_ref, dinv_ref, w_ref, g_ref, be_ref,
               y_ref):
    x = x_ref[...]
    dv = dinv_ref[...]
    s2 = s2_ref[0, pl.ds(0, N), :] + s2_ref[1, pl.ds(0, N), :]
    tx2 = -2.0 * (dv * s2) - x
    out = part_ref[...] + jnp.dot(tx2, w_ref[2],
                                  preferred_element_type=jnp.float32)
    mean = jnp.mean(out, axis=0, keepdims=True)
    var = jnp.mean((out - mean) ** 2, axis=0, keepdims=True)
    y_ref[...] = (out - mean) * lax.rsqrt(var + EPS) * g_ref[...] + be_ref[...]


_tcA = pl.pallas_call(
    _tcA_body,
    out_shape=(
        jax.ShapeDtypeStruct((NP, D), jnp.float32),
        jax.ShapeDtypeStruct((N, 1), jnp.float32),
    ),
)

_tcB = pl.pallas_call(
    _tcB_body,
    out_shape=(
        jax.ShapeDtypeStruct((N, D), jnp.float32),
        jax.ShapeDtypeStruct((NP, D), jnp.float32),
    ),
)

_tcC1 = pl.pallas_call(
    _tcC1_body,
    out_shape=jax.ShapeDtypeStruct((N, D), jnp.float32),
)

_tcC2 = pl.pallas_call(
    _tcC2_body,
    out_shape=jax.ShapeDtypeStruct((N, D), jnp.float32),
)


def kernel(x, edge_index, W, b, gamma, beta):
    src = edge_index[0]
    dst = edge_index[1]
    srcp = jnp.where(src == dst, N, src)

    npad = EPWP - EPW
    srcp_p = jnp.concatenate(
        [srcp.reshape(NW, EPW), jnp.full((NW, npad), N, jnp.int32)], axis=1
    ).reshape(NW, NCH, 1, CH)
    dst_p = jnp.concatenate(
        [dst.reshape(NW, EPW), jnp.zeros((NW, npad), jnp.int32)], axis=1
    ).reshape(NW, NCH, 1, CH)
    idx4 = jnp.concatenate([srcp_p, dst_p], axis=2)
    zrows = jnp.zeros((CH, D), jnp.float32)

    degp = _deg_kernel(src, dst)
    u0, dinv = _tcA(degp, x)
    s1 = _prop_kernel(u0, idx4, zrows)
    tx1, u1 = _tcB(s1, dinv)
    s2 = _prop_kernel(u1, idx4, zrows)
    part = _tcC1(x, tx1, W, b.reshape(1, D))
    return _tcC2(x, part, s2, dinv,
                 W, gamma.reshape(1, D), beta.reshape(1, D))


# speedup vs baseline: 1.2082x; 1.0050x over previous
import functools

import jax
import jax.numpy as jnp
from jax import lax
from jax.experimental import pallas as pl
from jax.experimental.pallas import tpu as pltpu
from jax.experimental.pallas import tpu_sc as plsc

N = 10000
E = 320000
D = 128
EPS = 1e-5
NP = N + 8

NC = 2
NS = 16
NW = NC * NS
EPW = E // NW
CH = 112
NCH = 90
EPWP = NCH * CH
NBUF = 3
ACCN = 10240
RPW = ACCN // NS

_mesh = plsc.VectorSubcoreMesh(
    core_axis_name="c", subcore_axis_name="s", num_cores=NC, num_subcores=NS
)

_sc_params = pltpu.CompilerParams(needs_layout_passes=False)


@functools.partial(
    pl.kernel,
    mesh=_mesh,
    out_type=jax.ShapeDtypeStruct((NW, N), jnp.float32),
    scratch_types=[
        pltpu.VMEM((EPW,), jnp.int32),
        pltpu.VMEM((EPW,), jnp.int32),
        pltpu.VMEM((N,), jnp.float32),
    ],
    compiler_params=_sc_params,
)
def _deg_kernel(src_hbm, dst_hbm, out_hbm, src_v, dst_v, acc_v):
    cid = lax.axis_index("c")
    sid = lax.axis_index("s")
    wid = sid * NC + cid
    base = wid * EPW
    pltpu.sync_copy(src_hbm.at[pl.ds(base, EPW)], src_v)
    pltpu.sync_copy(dst_hbm.at[pl.ds(base, EPW)], dst_v)

    zeros16 = jnp.zeros((16,), jnp.float32)

    def zero_body(i, carry):
        acc_v[pl.ds(i * 16, 16)] = zeros16
        return carry

    lax.fori_loop(0, N // 16, zero_body, 0)

    def edge_body(i, carry):
        s = src_v[pl.ds(i * 16, 16)]
        d = dst_v[pl.ds(i * 16, 16)]
        w = jnp.where(s != d, 1.0, 0.0).astype(jnp.float32)
        plsc.addupdate_scatter(acc_v, [s], w)
        return carry

    lax.fori_loop(0, EPW // 16, edge_body, 0)
    pltpu.sync_copy(acc_v, out_hbm.at[wid])


@functools.partial(
    pl.kernel,
    mesh=_mesh,
    out_type=jax.ShapeDtypeStruct((NC, ACCN, D), jnp.float32),
    scratch_types=[
        [pltpu.VMEM((2, CH), jnp.int32)] * NBUF,
        [pltpu.VMEM((2, CH), jnp.int32)] * NBUF,
        [pltpu.VMEM((CH, D), jnp.float32)] * NBUF,
        pltpu.VMEM_SHARED((ACCN, D), jnp.float32),
        [pltpu.SemaphoreType.DMA] * NBUF,
        [pltpu.SemaphoreType.DMA] * NBUF,
        [pltpu.SemaphoreType.DMA] * NBUF,
        [pltpu.SemaphoreType.DMA] * NBUF,
    ],
    compiler_params=_sc_params,
)
def _prop_kernel(u_hbm, idx_hbm, zrows_hbm, out_hbm,
                 idxa_v, idxb_v, rows_v, acc_sh, isema, isemb, gsem, ssem):
    cid = lax.axis_index("c")
    sid = lax.axis_index("s")
    wid = sid * NC + cid
    NR = NCH // NBUF

    for b in range(NBUF):
        pltpu.async_copy(idx_hbm.at[wid, b], idxa_v[b], isema[b])
        pltpu.async_copy(idx_hbm.at[wid, NBUF + b], idxb_v[b], isemb[b])

    pltpu.sync_copy(zrows_hbm, rows_v[0])
    for j in range(RPW // CH):
        pltpu.sync_copy(rows_v[0], acc_sh.at[pl.ds(sid * RPW + j * CH, CH)])
    rem = RPW - (RPW // CH) * CH
    if rem:
        pltpu.sync_copy(
            rows_v[0].at[pl.ds(0, rem)],
            acc_sh.at[pl.ds(sid * RPW + (RPW // CH) * CH, rem)])

    for b in range(NBUF):
        pltpu.make_async_copy(idx_hbm.at[wid, b], idxa_v[b], isema[b]).wait()
        pltpu.async_copy(u_hbm.at[idxa_v[b].at[0]], rows_v[b], gsem[b])
    plsc.subcore_barrier()

    def finish_launch(t, cur, csem, nxt, nsem):
        for b in range(NBUF):
            pltpu.make_async_copy(
                u_hbm.at[cur[b].at[0]], rows_v[b], gsem[b]).wait()
            pltpu.sync_copy(rows_v[b], acc_sh.at[cur[b].at[1]], add=True)

            @pl.when(t + 2 < NR)
            def _():
                pltpu.async_copy(
                    idx_hbm.at[wid, (t + 2) * NBUF + b], cur[b], csem[b])

            @pl.when(t + 1 < NR)
            def _():
                pltpu.make_async_copy(
                    idx_hbm.at[wid, (t + 1) * NBUF + b], nxt[b],
                    nsem[b]).wait()
                pltpu.async_copy(u_hbm.at[nxt[b].at[0]], rows_v[b], gsem[b])

    def pair_body(p, carry):
        finish_launch(2 * p, idxa_v, isema, idxb_v, isemb)
        finish_launch(2 * p + 1, idxb_v, isemb, idxa_v, isema)
        return carry

    lax.fori_loop(0, NR // 2, pair_body, 0)
    plsc.subcore_barrier()

    off = sid * RPW
    pltpu.sync_copy(acc_sh.at[pl.ds(off, RPW)],
                    out_hbm.at[cid, pl.ds(off, RPW)])


def _tcA_body(degp_ref, x_ref, u0_ref, dinv_ref):
    deg = jnp.sum(degp_ref[...], axis=0)
    dinv = jnp.where(deg > 0.0, lax.rsqrt(jnp.maximum(deg, 1.0)), 0.0)
    dv = dinv[:, None]
    dinv_ref[...] = dv
    u0_ref[pl.ds(0, N), :] = x_ref[...] * dv
    u0_ref[pl.ds(N, NP - N), :] = jnp.zeros((NP - N, D), jnp.float32)


def _tcB_body(s1_ref, dinv_ref, x_ref, w_ref, b_ref, u1_ref, part_ref):
    s = s1_ref[0, pl.ds(0, N), :] + s1_ref[1, pl.ds(0, N), :]
    dv = dinv_ref[...]
    tx1 = -(dv * s)
    u1_ref[pl.ds(0, N), :] = dv * tx1
    u1_ref[pl.ds(N, NP - N), :] = jnp.zeros((NP - N, D), jnp.float32)
    out = jnp.dot(x_ref[...], w_ref[0], preferred_element_type=jnp.float32)
    out += jnp.dot(tx1, w_ref[1], preferred_element_type=jnp.float32)
    part_ref[...] = out + b_ref[...]


def _tcC2_body(x_ref, part_ref, s2_ref, dinv_ref, w_ref, g_ref, be_ref,
               y_ref):
    x = x_ref[...]
    dv = dinv_ref[...]
    s2 = s2_ref[0, pl.ds(0, N), :] + s2_ref[1, pl.ds(0, N), :]
    tx2 = -2.0 * (dv * s2) - x
    out = part_ref[...] + jnp.dot(tx2, w_ref[2],
                                  preferred_element_type=jnp.float32)
    mean = jnp.mean(out, axis=0, keepdims=True)
    var = jnp.mean((out - mean) ** 2, axis=0, keepdims=True)
    y_ref[...] = (out - mean) * lax.rsqrt(var + EPS) * g_ref[...] + be_ref[...]


_tcA = pl.pallas_call(
    _tcA_body,
    out_shape=(
        jax.ShapeDtypeStruct((NP, D), jnp.float32),
        jax.ShapeDtypeStruct((N, 1), jnp.float32),
    ),
)

_tcB = pl.pallas_call(
    _tcB_body,
    out_shape=(
        jax.ShapeDtypeStruct((NP, D), jnp.float32),
        jax.ShapeDtypeStruct((N, D), jnp.float32),
    ),
)

_tcC2 = pl.pallas_call(
    _tcC2_body,
    out_shape=jax.ShapeDtypeStruct((N, D), jnp.float32),
)


def kernel(x, edge_index, W, b, gamma, beta):
    src = edge_index[0]
    dst = edge_index[1]
    srcp = jnp.where(src == dst, N, src)

    npad = EPWP - EPW
    srcp_p = jnp.concatenate(
        [srcp.reshape(NW, EPW), jnp.full((NW, npad), N, jnp.int32)], axis=1
    ).reshape(NW, NCH, 1, CH)
    dst_p = jnp.concatenate(
        [dst.reshape(NW, EPW), jnp.zeros((NW, npad), jnp.int32)], axis=1
    ).reshape(NW, NCH, 1, CH)
    idx4 = jnp.concatenate([srcp_p, dst_p], axis=2)
    zrows = jnp.zeros((CH, D), jnp.float32)

    degp = _deg_kernel(src, dst)
    u0, dinv = _tcA(degp, x)
    s1 = _prop_kernel(u0, idx4, zrows)
    u1, part = _tcB(s1, dinv, x, W, b.reshape(1, D))
    s2 = _prop_kernel(u1, idx4, zrows)
    return _tcC2(x, part, s2, dinv,
                 W, gamma.reshape(1, D), beta.reshape(1, D))
